# baseline (speedup 1.0000x reference)
import functools
import jax
import jax.numpy as jnp
from jax import lax
from jax.experimental import pallas as pl
from jax.experimental.pallas import tpu as pltpu
from jax.experimental.pallas import tpu_sc as plsc

ROWS = 8192
COLS = 128
GRID = 1
BLK = ROWS // GRID
G3 = 1
BLK3 = ROWS // G3

NW = 32
WROWS = ROWS // NW
SHIFT = 17
NBINS = 1 << 14


def _loss_bits_kernel(x_ref, lbl_ref, bits_ref):
    x0 = x_ref[0]
    x1 = x_ref[1]
    lbl = lbl_ref[...]
    diff = x1 - x0
    d = jnp.where(lbl == 0, diff, -diff)
    pe = jnp.maximum(d, 0.0) + jnp.log1p(jnp.exp(-jnp.abs(d)))
    pe = jnp.where(lbl < 0, 0.0, pe)
    bits_ref[...] = jax.lax.bitcast_convert_type(pe, jnp.int32)


def _sc_hist_kernel(bits_hbm, zeros_hbm, hist_hbm, vals_v, hist_v, sem):
    wid = lax.axis_index("s") * 2 + lax.axis_index("c")
    cp = pltpu.async_copy(bits_hbm.at[pl.ds(wid * WROWS, WROWS)], vals_v, sem)
    pltpu.sync_copy(zeros_hbm, hist_v)
    cp.wait()

    ones = jnp.ones((16,), jnp.int32)

    @plsc.parallel_loop(0, WROWS, 1, unroll=8)
    def hbody(i):
        for j in range(COLS // 16):
            v = vals_v[i, pl.ds(j * 16, 16)]
            b = lax.shift_right_logical(v, SHIFT)
            plsc.addupdate_scatter(hist_v, [b], ones)

    pltpu.sync_copy(hist_v, hist_hbm.at[wid])


def _merge_select_kernel(hist_ref, bits_ref, out_ref, sm_i, sm_f, *, keep):
    step = pl.program_id(0)

    @pl.when(step == 0)
    def _():
        m = jnp.sum(hist_ref[...], axis=0, keepdims=True).astype(jnp.float32)
        iota = lax.broadcasted_iota(jnp.int32, (1, NBINS), 1)
        keepf = jnp.float32(keep)

        def body(i, cur):
            cand = cur | (jnp.int32(1) << (jnp.int32(13) - i))
            cnt = jnp.sum(jnp.where(iota >= cand, m, 0.0))
            return jnp.where(cnt >= keepf, cand, cur)

        bstar = lax.fori_loop(0, 14, body, jnp.int32(0))
        sm_i[0] = (bstar + 1) << SHIFT
        sm_i[1] = 0
        tmid_bits = (bstar << SHIFT) | (1 << (SHIFT - 1))
        sm_f[0] = jax.lax.bitcast_convert_type(tmid_bits, jnp.float32)
        sm_f[1] = 0.0

    @pl.when(step > 0)
    def _():
        bits = bits_ref[...]
        thr = sm_i[0]
        gt = bits >= thr
        pe = jax.lax.bitcast_convert_type(bits, jnp.float32)
        sm_f[1] += jnp.sum(jnp.where(gt, pe, 0.0))
        sm_i[1] += jnp.sum(gt.astype(jnp.int32))

    @pl.when(step == G3)
    def _():
        r = (keep - sm_i[1]).astype(jnp.float32)
        out_ref[0, 0] = (sm_f[1] + r * sm_f[0]) / keep


def kernel(class_out, label):
    n = label.shape[0]
    keep = int(n * 0.7)
    xt = jnp.transpose(class_out.astype(jnp.float32)).reshape(2, ROWS, COLS)
    lbl = label.astype(jnp.int32).reshape(ROWS, COLS)

    bits = pl.pallas_call(
        _loss_bits_kernel,
        grid=(GRID,),
        in_specs=[
            pl.BlockSpec((2, BLK, COLS), lambda i: (0, i, 0)),
            pl.BlockSpec((BLK, COLS), lambda i: (i, 0)),
        ],
        out_specs=pl.BlockSpec((BLK, COLS), lambda i: (i, 0)),
        out_shape=jax.ShapeDtypeStruct((ROWS, COLS), jnp.int32),
    )(xt, lbl)

    mesh = plsc.VectorSubcoreMesh(core_axis_name="c", subcore_axis_name="s")
    hists = pl.kernel(
        _sc_hist_kernel,
        out_type=jax.ShapeDtypeStruct((NW, NBINS), jnp.int32),
        mesh=mesh,
        compiler_params=pltpu.CompilerParams(
            needs_layout_passes=False, use_tc_tiling_on_sc=True),
        scratch_types=[
            pltpu.VMEM((WROWS, COLS), jnp.int32),
            pltpu.VMEM((NBINS,), jnp.int32),
            pltpu.SemaphoreType.DMA,
        ],
    )(bits, jnp.zeros((NBINS,), jnp.int32))

    out = pl.pallas_call(
        functools.partial(_merge_select_kernel, keep=keep),
        grid=(G3 + 1,),
        in_specs=[
            pl.BlockSpec((NW, NBINS), lambda i: (0, 0)),
            pl.BlockSpec((BLK3, COLS),
                         lambda i: (jnp.maximum(i - 1, 0), 0)),
        ],
        out_specs=pl.BlockSpec(
            (1, 1), lambda i: (0, 0), memory_space=pltpu.SMEM),
        out_shape=jax.ShapeDtypeStruct((1, 1), jnp.float32),
        scratch_shapes=[
            pltpu.SMEM((2,), jnp.int32),
            pltpu.SMEM((2,), jnp.float32),
        ],
    )(hists, bits)
    return out[0, 0]


# speedup vs baseline: 1.0331x; 1.0331x over previous
import functools
import jax
import jax.numpy as jnp
from jax import lax
from jax.experimental import pallas as pl
from jax.experimental.pallas import tpu as pltpu
from jax.experimental.pallas import tpu_sc as plsc

ROWS = 8192
COLS = 128
GRID = 2
BLK = ROWS // GRID
G3 = 2
BLK3 = ROWS // G3

NW = 32
WROWS = ROWS // NW
SHIFT = 17
NBINS = 1 << 14


def _loss_bits_kernel(x_ref, lbl_ref, bits_ref):
    x0 = x_ref[0]
    x1 = x_ref[1]
    lbl = lbl_ref[...]
    diff = x1 - x0
    d = jnp.where(lbl == 0, diff, -diff)
    pe = jnp.maximum(d, 0.0) + jnp.log1p(jnp.exp(-jnp.abs(d)))
    pe = jnp.where(lbl < 0, 0.0, pe)
    bits_ref[...] = jax.lax.bitcast_convert_type(pe, jnp.int32)


def _sc_hist_kernel(bits_hbm, zeros_hbm, hist_hbm, vals_v, hist_v, sem):
    wid = lax.axis_index("s") * 2 + lax.axis_index("c")
    cp = pltpu.async_copy(bits_hbm.at[pl.ds(wid * WROWS, WROWS)], vals_v, sem)
    pltpu.sync_copy(zeros_hbm, hist_v)
    cp.wait()

    ones = jnp.ones((16,), jnp.int32)

    @plsc.parallel_loop(0, WROWS, 1, unroll=8)
    def hbody(i):
        for j in range(COLS // 16):
            v = vals_v[i, pl.ds(j * 16, 16)]
            b = lax.shift_right_logical(v, SHIFT)
            plsc.addupdate_scatter(hist_v, [b], ones)

    pltpu.sync_copy(hist_v, hist_hbm.at[wid])


def _merge_select_kernel(hist_ref, bits_ref, out_ref, sm_i, sm_f, *, keep):
    step = pl.program_id(0)

    @pl.when(step == 0)
    def _():
        m = jnp.sum(hist_ref[...], axis=0, keepdims=True).astype(jnp.float32)
        iota = lax.broadcasted_iota(jnp.int32, (1, NBINS), 1)
        keepf = jnp.float32(keep)

        def body(i, cur):
            cand = cur | (jnp.int32(1) << (jnp.int32(13) - i))
            cnt = jnp.sum(jnp.where(iota >= cand, m, 0.0))
            return jnp.where(cnt >= keepf, cand, cur)

        bstar = lax.fori_loop(0, 14, body, jnp.int32(0))
        sm_i[0] = (bstar + 1) << SHIFT
        sm_i[1] = 0
        tmid_bits = (bstar << SHIFT) | (1 << (SHIFT - 1))
        sm_f[0] = jax.lax.bitcast_convert_type(tmid_bits, jnp.float32)
        sm_f[1] = 0.0

    @pl.when(step > 0)
    def _():
        bits = bits_ref[...]
        thr = sm_i[0]
        gt = bits >= thr
        pe = jax.lax.bitcast_convert_type(bits, jnp.float32)
        sm_f[1] += jnp.sum(jnp.where(gt, pe, 0.0))
        sm_i[1] += jnp.sum(gt.astype(jnp.int32))

    @pl.when(step == G3)
    def _():
        r = (keep - sm_i[1]).astype(jnp.float32)
        out_ref[0, 0] = (sm_f[1] + r * sm_f[0]) / keep


def kernel(class_out, label):
    n = label.shape[0]
    keep = int(n * 0.7)
    xt = jnp.transpose(class_out.astype(jnp.float32)).reshape(2, ROWS, COLS)
    lbl = label.astype(jnp.int32).reshape(ROWS, COLS)

    bits = pl.pallas_call(
        _loss_bits_kernel,
        grid=(GRID,),
        in_specs=[
            pl.BlockSpec((2, BLK, COLS), lambda i: (0, i, 0)),
            pl.BlockSpec((BLK, COLS), lambda i: (i, 0)),
        ],
        out_specs=pl.BlockSpec((BLK, COLS), lambda i: (i, 0)),
        out_shape=jax.ShapeDtypeStruct((ROWS, COLS), jnp.int32),
    )(xt, lbl)

    mesh = plsc.VectorSubcoreMesh(core_axis_name="c", subcore_axis_name="s")
    hists = pl.kernel(
        _sc_hist_kernel,
        out_type=jax.ShapeDtypeStruct((NW, NBINS), jnp.int32),
        mesh=mesh,
        compiler_params=pltpu.CompilerParams(
            needs_layout_passes=False, use_tc_tiling_on_sc=True),
        scratch_types=[
            pltpu.VMEM((WROWS, COLS), jnp.int32),
            pltpu.VMEM((NBINS,), jnp.int32),
            pltpu.SemaphoreType.DMA,
        ],
    )(bits, jnp.zeros((NBINS,), jnp.int32))

    out = pl.pallas_call(
        functools.partial(_merge_select_kernel, keep=keep),
        grid=(G3 + 1,),
        in_specs=[
            pl.BlockSpec((NW, NBINS), lambda i: (0, 0)),
            pl.BlockSpec((BLK3, COLS),
                         lambda i: (jnp.maximum(i - 1, 0), 0)),
        ],
        out_specs=pl.BlockSpec(
            (1, 1), lambda i: (0, 0), memory_space=pltpu.SMEM),
        out_shape=jax.ShapeDtypeStruct((1, 1), jnp.float32),
        scratch_shapes=[
            pltpu.SMEM((2,), jnp.int32),
            pltpu.SMEM((2,), jnp.float32),
        ],
    )(hists, bits)
    return out[0, 0]
